# baseline (speedup 1.0000x reference)
import jax
import jax.numpy as jnp
from jax import lax
from jax.experimental import pallas as pl
from jax.experimental.pallas import tpu as pltpu
from jax.experimental.pallas import tpu_sc as plsc

B = 4096
L = 200
D = 64
C = 2
V = 1000000
V8 = V // 8

NC = 2
NS = 16
NW = NC * NS
BPW = B // NW
CHUNK = 112
ROWS_I = 2 * BPW
NBUF = 4


BLKV = 8192


def _proj_body(t_ref, w_ref, q_ref):
    d = lax.dot_general(w_ref[...], t_ref[...], (((1,), (0,)), ((), ())),
                        preferred_element_type=jnp.float32)
    u0 = lax.bitcast_convert_type(
        d[0, :].astype(jnp.bfloat16), jnp.uint16).astype(jnp.uint32)
    u1 = lax.bitcast_convert_type(
        d[1, :].astype(jnp.bfloat16), jnp.uint16).astype(jnp.uint32)
    q_ref[...] = lax.bitcast_convert_type((u0 << 16) | u1, jnp.float32)


_project = pl.pallas_call(
    _proj_body,
    grid=(pl.cdiv(V, BLKV),),
    in_specs=[
        pl.BlockSpec((D, BLKV), lambda i: (0, i)),
        pl.BlockSpec((C, D), lambda i: (0, 0)),
    ],
    out_specs=pl.BlockSpec((BLKV,), lambda i: (i,)),
    out_shape=jax.ShapeDtypeStruct((V,), jnp.float32),
)


def _sc_body(hi_hbm, lo_hbm, q_hbm, out_hbm,
             hi_v, lo_v, rows_v, out_v, s0, s1, s2, s3):
    sems = (s0, s1, s2, s3)
    wid = lax.axis_index("s") * NC + lax.axis_index("c")
    base = wid * BPW
    pltpu.sync_copy(hi_hbm.at[pl.ds(base * 2, ROWS_I)], hi_v)
    pltpu.sync_copy(lo_hbm.at[pl.ds(base * 2, ROWS_I)], lo_v)

    z = jnp.zeros((16,), jnp.float32)
    lane = lax.iota(jnp.int32, 16)

    def gather(c, j):
        return pltpu.make_async_copy(
            q_hbm.at[hi_v.at[c]], rows_v.at[j], sems[j])

    def start(c, j):
        gather(c, j).start()

    def wait(c, j):
        gather(c, j).wait()

    for j in range(NBUF):
        start(j, j)

    hi_mask = jnp.full((16,), 0xFFFF0000, jnp.uint32)

    def accumulate(c, j, accs):
        def tok(t, accs):
            a0, a1 = accs
            col = lo_v[c, pl.ds(16 * t, 16)]
            row = 16 * t + lane
            x = plsc.bitcast(
                plsc.load_gather(rows_v.at[j], [row, col]), jnp.uint32)
            a0 = a0 + plsc.bitcast(x & hi_mask, jnp.float32)
            a1 = a1 + plsc.bitcast(x << 16, jnp.float32)
            return (a0, a1)
        return lax.fori_loop(0, CHUNK // 16, tok, accs, unroll=7)

    def pair_body(bb, _):
        c0 = NBUF * bb
        accs = (z, z)
        for j in range(NBUF):
            c = c0 + j
            wait(c, j)
            accs = accumulate(c, j, accs)
            nxt = jnp.minimum(c + NBUF, ROWS_I - 1)
            start(nxt, j)
            if j % 2 == 1:
                out_v[(NBUF // 2) * bb + j // 2, pl.ds(0, 16)] = accs[0]
                out_v[(NBUF // 2) * bb + j // 2, pl.ds(16, 16)] = accs[1]
                accs = (z, z)
        return 0

    lax.fori_loop(0, ROWS_I // NBUF, pair_body, 0)
    for j in range(NBUF):
        wait(ROWS_I - 1, j)
    pltpu.sync_copy(out_v, out_hbm.at[pl.ds(base, BPW)])


_SC_CACHE = {}


def _sc_gather_sum_fn():
    if "k" not in _SC_CACHE:
        _SC_CACHE["k"] = pl.kernel(
            _sc_body,
            out_type=jax.ShapeDtypeStruct((B, 2 * 16), jnp.float32),
            mesh=plsc.VectorSubcoreMesh(
                core_axis_name="c", subcore_axis_name="s",
                num_cores=NC, num_subcores=NS,
            ),
            scratch_types=[
                pltpu.VMEM((ROWS_I, CHUNK), jnp.int32),
                pltpu.VMEM((ROWS_I, CHUNK), jnp.int32),
                pltpu.VMEM((NBUF, CHUNK, 8), jnp.float32),
                pltpu.VMEM((BPW, 2 * 16), jnp.float32),
            ] + [pltpu.SemaphoreType.DMA] * NBUF,
            compiler_params=pltpu.CompilerParams(
                use_tc_tiling_on_sc=False, needs_layout_passes=False),
        )
    return _SC_CACHE["k"]


BB = 512


def _head_body(ids_ref, sums_ref, b_ref, out_ref):
    ids = ids_ref[...]
    cnt = jnp.sum((ids != 0).astype(jnp.float32), axis=1, keepdims=True)
    s0 = jnp.sum(sums_ref[:, 0:16], axis=1, keepdims=True)
    s1 = jnp.sum(sums_ref[:, 16:32], axis=1, keepdims=True)
    s = jnp.concatenate([s0, s1], axis=1)
    out_ref[...] = s / (cnt + 1e-8) + b_ref[...]


_head = pl.pallas_call(
    _head_body,
    grid=(B // BB,),
    in_specs=[
        pl.BlockSpec((BB, L), lambda i: (i, 0)),
        pl.BlockSpec((BB, 2 * 16), lambda i: (i, 0)),
        pl.BlockSpec((1, C), lambda i: (0, 0)),
    ],
    out_specs=pl.BlockSpec((BB, C), lambda i: (i, 0)),
    out_shape=jax.ShapeDtypeStruct((B, C), jnp.float32),
)


def kernel(input_ids, table, W, b):
    ids = input_ids.astype(jnp.int32)
    q = _project(table.T, W)
    q8 = jnp.tile(q, 8)
    ids_pad = jnp.pad(ids, ((0, 0), (0, 2 * CHUNK - L))).reshape(B * 2, CHUNK)
    lo = ids_pad & 7
    hi = lo * V8 + (ids_pad >> 3)
    sums = _sc_gather_sum_fn()(hi, lo, q8.reshape(8 * V8, 8))
    return _head(ids, sums, b.reshape(1, C))


# speedup vs baseline: 5.2208x; 5.2208x over previous
import jax
import jax.numpy as jnp
from jax import lax
from jax.experimental import pallas as pl
from jax.experimental.pallas import tpu as pltpu
from jax.experimental.pallas import tpu_sc as plsc

B = 4096
L = 200
D = 64
C = 2
V = 1000000
V8 = V // 8

NC = 2
NS = 16
NW = NC * NS
BPW = B // NW
CHUNK = 100
ROWS_I = 2 * BPW
NBUF = 4


BLKV = 8192


def _proj_body(t_ref, w_ref, q_ref):
    d = lax.dot_general(w_ref[...], t_ref[...], (((1,), (0,)), ((), ())),
                        preferred_element_type=jnp.float32)
    u0 = lax.bitcast_convert_type(
        d[0, :].astype(jnp.bfloat16), jnp.uint16).astype(jnp.uint32)
    u1 = lax.bitcast_convert_type(
        d[1, :].astype(jnp.bfloat16), jnp.uint16).astype(jnp.uint32)
    q_ref[...] = lax.bitcast_convert_type((u0 << 16) | u1, jnp.float32)


_project = pl.pallas_call(
    _proj_body,
    grid=(pl.cdiv(V, BLKV),),
    in_specs=[
        pl.BlockSpec((D, BLKV), lambda i: (0, i)),
        pl.BlockSpec((C, D), lambda i: (0, 0)),
    ],
    out_specs=pl.BlockSpec((BLKV,), lambda i: (i,)),
    out_shape=jax.ShapeDtypeStruct((V,), jnp.float32),
)


def _sc_body(hi_hbm, lo_hbm, q_hbm, out_hbm,
             hi_v, lo_v, rows_v, out_v, s0, s1, s2, s3):
    sems = (s0, s1, s2, s3)
    wid = lax.axis_index("s") * NC + lax.axis_index("c")
    base = wid * BPW
    pltpu.sync_copy(hi_hbm.at[pl.ds(base * 2, ROWS_I)], hi_v)
    pltpu.sync_copy(lo_hbm.at[pl.ds(base * 2, ROWS_I)], lo_v)

    z = jnp.zeros((16,), jnp.float32)
    lane = lax.iota(jnp.int32, 16)

    def gather(c, j):
        return pltpu.make_async_copy(
            q_hbm.at[hi_v.at[c]], rows_v.at[j], sems[j])

    def start(c, j):
        gather(c, j).start()

    def wait(c, j):
        gather(c, j).wait()

    for j in range(NBUF):
        start(j, j)

    hi_mask = jnp.full((16,), 0xFFFF0000, jnp.uint32)

    tail_m = lane >= 12

    def accumulate(c, j, accs):
        def tok1(base, accs, mask=None):
            a0, a1 = accs
            col = lo_v[c, pl.ds(base, 16)]
            row = base + lane
            x = plsc.bitcast(
                plsc.load_gather(rows_v.at[j], [row, col], mask=mask),
                jnp.uint32)
            if mask is not None:
                x = jnp.where(mask, x, 0)
            a0 = a0 + plsc.bitcast(x & hi_mask, jnp.float32)
            a1 = a1 + plsc.bitcast(x << 16, jnp.float32)
            return (a0, a1)
        accs = lax.fori_loop(
            0, CHUNK // 16, lambda t, a: tok1(16 * t, a), accs, unroll=6)
        return tok1(84, accs, mask=tail_m)

    def pair_body(bb, _):
        c0 = NBUF * bb
        accs = (z, z)
        for j in range(NBUF):
            c = c0 + j
            wait(c, j)
            accs = accumulate(c, j, accs)
            nxt = jnp.minimum(c + NBUF, ROWS_I - 1)
            start(nxt, j)
            if j % 2 == 1:
                out_v[(NBUF // 2) * bb + j // 2, pl.ds(0, 16)] = accs[0]
                out_v[(NBUF // 2) * bb + j // 2, pl.ds(16, 16)] = accs[1]
                accs = (z, z)
        return 0

    lax.fori_loop(0, ROWS_I // NBUF, pair_body, 0)
    for j in range(NBUF):
        wait(ROWS_I - 1, j)
    pltpu.sync_copy(out_v, out_hbm.at[pl.ds(base, BPW)])


_SC_CACHE = {}


def _sc_gather_sum_fn():
    if "k" not in _SC_CACHE:
        _SC_CACHE["k"] = pl.kernel(
            _sc_body,
            out_type=jax.ShapeDtypeStruct((B, 2 * 16), jnp.float32),
            mesh=plsc.VectorSubcoreMesh(
                core_axis_name="c", subcore_axis_name="s",
                num_cores=NC, num_subcores=NS,
            ),
            scratch_types=[
                pltpu.VMEM((ROWS_I, CHUNK), jnp.int32),
                pltpu.VMEM((ROWS_I, CHUNK), jnp.int32),
                pltpu.VMEM((NBUF, CHUNK, 8), jnp.float32),
                pltpu.VMEM((BPW, 2 * 16), jnp.float32),
            ] + [pltpu.SemaphoreType.DMA] * NBUF,
            compiler_params=pltpu.CompilerParams(
                use_tc_tiling_on_sc=False, needs_layout_passes=False),
        )
    return _SC_CACHE["k"]


BB = 512


def _head_body(ids_ref, sums_ref, b_ref, out_ref):
    ids = ids_ref[...]
    cnt = jnp.sum((ids != 0).astype(jnp.float32), axis=1, keepdims=True)
    s0 = jnp.sum(sums_ref[:, 0:16], axis=1, keepdims=True)
    s1 = jnp.sum(sums_ref[:, 16:32], axis=1, keepdims=True)
    s = jnp.concatenate([s0, s1], axis=1)
    out_ref[...] = s / (cnt + 1e-8) + b_ref[...]


_head = pl.pallas_call(
    _head_body,
    grid=(B // BB,),
    in_specs=[
        pl.BlockSpec((BB, L), lambda i: (i, 0)),
        pl.BlockSpec((BB, 2 * 16), lambda i: (i, 0)),
        pl.BlockSpec((1, C), lambda i: (0, 0)),
    ],
    out_specs=pl.BlockSpec((BB, C), lambda i: (i, 0)),
    out_shape=jax.ShapeDtypeStruct((B, C), jnp.float32),
)


def kernel(input_ids, table, W, b):
    ids = input_ids.astype(jnp.int32)
    q = _project(table.T, W)
    ids2 = ids.reshape(B * 2, CHUNK)
    sums = _sc_gather_sum_fn()(ids2 >> 3, ids2 & 7, q.reshape(V8, 8))
    return _head(ids, sums, b.reshape(1, C))


# speedup vs baseline: 6.4390x; 1.2333x over previous
import jax
import jax.numpy as jnp
from jax import lax
from jax.experimental import pallas as pl
from jax.experimental.pallas import tpu as pltpu
from jax.experimental.pallas import tpu_sc as plsc

B = 4096
L = 200
D = 64
C = 2
V = 1000000
V8 = V // 8

NC = 2
NS = 16
NW = NC * NS
BPW = B // NW
CHUNK = 100
ROWS_I = 2 * BPW
NBUF = 4


BLKV = 32768


def _proj_body(t_ref, w_ref, q_ref):
    d = lax.dot_general(w_ref[...], t_ref[...], (((1,), (0,)), ((), ())),
                        preferred_element_type=jnp.float32)
    u0 = lax.bitcast_convert_type(
        d[0, :].astype(jnp.bfloat16), jnp.uint16).astype(jnp.uint32)
    u1 = lax.bitcast_convert_type(
        d[1, :].astype(jnp.bfloat16), jnp.uint16).astype(jnp.uint32)
    q_ref[...] = lax.bitcast_convert_type((u0 << 16) | u1, jnp.float32)


_project = pl.pallas_call(
    _proj_body,
    grid=(pl.cdiv(V, BLKV),),
    in_specs=[
        pl.BlockSpec((D, BLKV), lambda i: (0, i)),
        pl.BlockSpec((C, D), lambda i: (0, 0)),
    ],
    out_specs=pl.BlockSpec((BLKV,), lambda i: (i,)),
    out_shape=jax.ShapeDtypeStruct((V,), jnp.float32),
)


def _sc_body(hi_hbm, lo_hbm, q_hbm, out_hbm,
             hi_v, lo_v, rows_v, out_v, s0, s1, s2, s3):
    sems = (s0, s1, s2, s3)
    wid = lax.axis_index("s") * NC + lax.axis_index("c")
    base = wid * BPW
    pltpu.sync_copy(hi_hbm.at[pl.ds(base * 2, ROWS_I)], hi_v)
    pltpu.sync_copy(lo_hbm.at[pl.ds(base * 2, ROWS_I)], lo_v)

    z = jnp.zeros((16,), jnp.float32)
    lane = lax.iota(jnp.int32, 16)

    def gather(c, j):
        return pltpu.make_async_copy(
            q_hbm.at[hi_v.at[c]], rows_v.at[j], sems[j])

    def start(c, j):
        gather(c, j).start()

    def wait(c, j):
        gather(c, j).wait()

    for j in range(NBUF):
        start(j, j)

    hi_mask = jnp.full((16,), 0xFFFF0000, jnp.uint32)

    tail_m = lane >= 12

    def accumulate(c, j, accs):
        def tok1(base, accs, mask=None):
            a0, a1 = accs
            col = lo_v[c, pl.ds(base, 16)]
            row = base + lane
            x = plsc.bitcast(
                plsc.load_gather(rows_v.at[j], [row, col], mask=mask),
                jnp.uint32)
            if mask is not None:
                x = jnp.where(mask, x, 0)
            a0 = a0 + plsc.bitcast(x & hi_mask, jnp.float32)
            a1 = a1 + plsc.bitcast(x << 16, jnp.float32)
            return (a0, a1)
        accs = lax.fori_loop(
            0, CHUNK // 16, lambda t, a: tok1(16 * t, a), accs, unroll=6)
        return tok1(84, accs, mask=tail_m)

    def pair_body(bb, _):
        c0 = NBUF * bb
        accs = (z, z)
        for j in range(NBUF):
            c = c0 + j
            wait(c, j)
            accs = accumulate(c, j, accs)
            nxt = jnp.minimum(c + NBUF, ROWS_I - 1)
            start(nxt, j)
            if j % 2 == 1:
                out_v[(NBUF // 2) * bb + j // 2, pl.ds(0, 16)] = accs[0]
                out_v[(NBUF // 2) * bb + j // 2, pl.ds(16, 16)] = accs[1]
                accs = (z, z)
        return 0

    lax.fori_loop(0, ROWS_I // NBUF, pair_body, 0)
    for j in range(NBUF):
        wait(ROWS_I - 1, j)
    pltpu.sync_copy(out_v, out_hbm.at[pl.ds(base, BPW)])


_SC_CACHE = {}


def _sc_gather_sum_fn():
    if "k" not in _SC_CACHE:
        _SC_CACHE["k"] = pl.kernel(
            _sc_body,
            out_type=jax.ShapeDtypeStruct((B, 2 * 16), jnp.float32),
            mesh=plsc.VectorSubcoreMesh(
                core_axis_name="c", subcore_axis_name="s",
                num_cores=NC, num_subcores=NS,
            ),
            scratch_types=[
                pltpu.VMEM((ROWS_I, CHUNK), jnp.int32),
                pltpu.VMEM((ROWS_I, CHUNK), jnp.int32),
                pltpu.VMEM((NBUF, CHUNK, 8), jnp.float32),
                pltpu.VMEM((BPW, 2 * 16), jnp.float32),
            ] + [pltpu.SemaphoreType.DMA] * NBUF,
            compiler_params=pltpu.CompilerParams(
                use_tc_tiling_on_sc=False, needs_layout_passes=False),
        )
    return _SC_CACHE["k"]


BB = 512


def _head_body(ids_ref, sums_ref, b_ref, out_ref):
    ids = ids_ref[...]
    cnt = jnp.sum((ids != 0).astype(jnp.float32), axis=1, keepdims=True)
    s0 = jnp.sum(sums_ref[:, 0:16], axis=1, keepdims=True)
    s1 = jnp.sum(sums_ref[:, 16:32], axis=1, keepdims=True)
    s = jnp.concatenate([s0, s1], axis=1)
    out_ref[...] = s / (cnt + 1e-8) + b_ref[...]


_head = pl.pallas_call(
    _head_body,
    grid=(B // BB,),
    in_specs=[
        pl.BlockSpec((BB, L), lambda i: (i, 0)),
        pl.BlockSpec((BB, 2 * 16), lambda i: (i, 0)),
        pl.BlockSpec((1, C), lambda i: (0, 0)),
    ],
    out_specs=pl.BlockSpec((BB, C), lambda i: (i, 0)),
    out_shape=jax.ShapeDtypeStruct((B, C), jnp.float32),
)


def kernel(input_ids, table, W, b):
    ids = input_ids.astype(jnp.int32)
    q = _project(table.T, W)
    ids2 = ids.reshape(B * 2, CHUNK)
    sums = _sc_gather_sum_fn()(ids2 >> 3, ids2 & 7, q.reshape(V8, 8))
    return _head(ids, sums, b.reshape(1, C))
